# speedup vs baseline: 3.8927x; 3.8927x over previous
import functools

import jax
import jax.numpy as jnp
from jax import lax
from jax.experimental import pallas as pl
from jax.experimental.pallas import tpu as pltpu

P = 128
B = 128
H = 256
ALPHA = 0.5
CMIX = (1.0 - ALPHA) / P
TINY = float(jnp.finfo(jnp.float32).tiny)
CHUNK = 8
GRID = P // CHUNK


def _threefry_bits(lo):
    k0 = jnp.uint32(0)
    k1 = jnp.uint32(42)
    k2 = jnp.uint32(0 ^ 42 ^ 0x1BD11BDA)

    def rounds(x0, x1, rots):
        for r in rots:
            x0 = x0 + x1
            x1 = (x1 << jnp.uint32(r)) | (x1 >> jnp.uint32(32 - r))
            x1 = x1 ^ x0
        return x0, x1

    r0 = (13, 15, 26, 6)
    r1 = (17, 29, 16, 24)
    x0 = jnp.full_like(lo, k0)
    x1 = lo + k1
    x0, x1 = rounds(x0, x1, r0)
    x0 = x0 + k1
    x1 = x1 + k2 + jnp.uint32(1)
    x0, x1 = rounds(x0, x1, r1)
    x0 = x0 + k2
    x1 = x1 + k0 + jnp.uint32(2)
    x0, x1 = rounds(x0, x1, r0)
    x0 = x0 + k0
    x1 = x1 + k1 + jnp.uint32(3)
    x0, x1 = rounds(x0, x1, r1)
    x0 = x0 + k1
    x1 = x1 + k2 + jnp.uint32(4)
    x0, x1 = rounds(x0, x1, r0)
    x0 = x0 + k2
    x1 = x1 + k0 + jnp.uint32(5)
    return x0 ^ x1


def _resample_kernel(prob_pb_ref, table_ref, probv_ref, pout_ref, pnew_ref,
                     logits_s, pn_s):
    c = pl.program_id(0)

    @pl.when(c == 0)
    def _():
        logits_s[...] = jnp.log(ALPHA * jnp.exp(prob_pb_ref[...]) + CMIX)

    logits = logits_s[...]
    pidx = lax.broadcasted_iota(jnp.int32, (P, B), 0)
    lane_b = lax.broadcasted_iota(jnp.uint32, (P, B), 1)
    subl_p = lax.broadcasted_iota(jnp.uint32, (P, B), 0)

    for k in range(CHUNK):
        i = c * CHUNK + k
        flat = jnp.uint32(P * B) * i.astype(jnp.uint32) + lane_b * jnp.uint32(P) + subl_p
        bits = _threefry_bits(flat)
        fb = (bits >> jnp.uint32(9)) | jnp.uint32(0x3F800000)
        u = lax.bitcast_convert_type(fb, jnp.float32) - jnp.float32(1.0)
        u = u * jnp.float32(1.0 - TINY) + jnp.float32(TINY)
        u = jnp.maximum(jnp.float32(TINY), u)
        g = -jnp.log(-jnp.log(u))
        scores = g + logits
        m = jnp.max(scores, axis=0, keepdims=True)
        samp = jnp.min(jnp.where(scores == m, pidx, jnp.int32(P)),
                       axis=0, keepdims=True)
        onehot = (pidx == samp).astype(jnp.float32)
        pout_ref[pl.ds(k * B, B), :] = lax.dot_general(
            onehot, table_ref[...], (((0,), (0,)), ((), ())),
            preferred_element_type=jnp.float32)
        pg = jnp.dot(probv_ref[...], onehot,
                     preferred_element_type=jnp.float32)
        e = jnp.exp(pg)
        pn_s[pl.ds(i, 1), :] = jnp.log(e / (ALPHA * e + CMIX))

    @pl.when(c == GRID - 1)
    def _():
        pn = pn_s[...]
        m2 = jnp.max(pn, axis=0, keepdims=True)
        lse = jnp.log(jnp.sum(jnp.exp(pn - m2), axis=0, keepdims=True)) + m2
        pnew_ref[...] = pn - lse


@functools.partial(jax.jit, static_argnames=("interpret",))
def kernel(particles, prob, interpret=False):
    prob_pb = prob.reshape(P, B)
    table = particles[:P]
    probv = prob.reshape(1, -1)[:, :P]
    particles_new, prob_new = pl.pallas_call(
        _resample_kernel,
        grid=(GRID,),
        in_specs=[
            pl.BlockSpec((P, B), lambda c: (0, 0)),
            pl.BlockSpec((P, H), lambda c: (0, 0)),
            pl.BlockSpec((1, P), lambda c: (0, 0)),
        ],
        out_specs=[
            pl.BlockSpec((CHUNK * B, H), lambda c: (c, 0)),
            pl.BlockSpec((P, B), lambda c: (0, 0)),
        ],
        out_shape=[
            jax.ShapeDtypeStruct((P * B, H), jnp.float32),
            jax.ShapeDtypeStruct((P, B), jnp.float32),
        ],
        scratch_shapes=[
            pltpu.VMEM((P, B), jnp.float32),
            pltpu.VMEM((P, B), jnp.float32),
        ],
        interpret=interpret,
    )(prob_pb, table, probv)
    return particles_new, prob_new


# speedup vs baseline: 3.9896x; 1.0249x over previous
import functools

import jax
import jax.numpy as jnp
from jax import lax
from jax.experimental import pallas as pl
from jax.experimental.pallas import tpu as pltpu

P = 128
B = 128
H = 256
ALPHA = 0.5
CMIX = (1.0 - ALPHA) / P
TINY = float(jnp.finfo(jnp.float32).tiny)
CHUNK = 8
GRID = P // CHUNK


_KS1 = 42
_KS2 = 42 ^ 0x1BD11BDA


def _threefry_bits(x1):
    def rounds(x0, x1, rots, first=False):
        for r in rots:
            x0 = x1 if first else x0 + x1
            first = False
            x1 = (x1 << jnp.uint32(r)) | (x1 >> jnp.uint32(32 - r))
            x1 = x1 ^ x0
        return x0, x1

    r0 = (13, 15, 26, 6)
    r1 = (17, 29, 16, 24)
    x0, x1 = rounds(None, x1, r0, first=True)
    x0 = x0 + jnp.uint32(_KS1)
    x1 = x1 + jnp.uint32((_KS2 + 1) & 0xFFFFFFFF)
    x0, x1 = rounds(x0, x1, r1)
    x0 = x0 + jnp.uint32(_KS2)
    x1 = x1 + jnp.uint32(2)
    x0, x1 = rounds(x0, x1, r0)
    x1 = x1 + jnp.uint32(_KS1 + 3)
    x0, x1 = rounds(x0, x1, r1)
    x0 = x0 + jnp.uint32(_KS1)
    x1 = x1 + jnp.uint32((_KS2 + 4) & 0xFFFFFFFF)
    x0, x1 = rounds(x0, x1, r0)
    x0 = x0 + jnp.uint32(_KS2)
    x1 = x1 + jnp.uint32(5)
    return x0 ^ x1


def _resample_kernel(prob_pb_ref, table_ref, probv_ref, pout_ref, pnew_ref,
                     logits_s, pn_s):
    c = pl.program_id(0)

    @pl.when(c == 0)
    def _():
        logits_s[...] = jnp.log(ALPHA * jnp.exp(prob_pb_ref[...]) + CMIX)

    logits = logits_s[...]
    pidx = lax.broadcasted_iota(jnp.int32, (P, B), 0)
    lane_b = lax.broadcasted_iota(jnp.uint32, (P, B), 1)
    subl_p = lax.broadcasted_iota(jnp.uint32, (P, B), 0)
    lbp = lane_b * jnp.uint32(P) + subl_p

    for k in range(CHUNK):
        i = c * CHUNK + k
        x1 = lbp + (jnp.uint32(P * B) * i.astype(jnp.uint32) + jnp.uint32(_KS1))
        bits = _threefry_bits(x1)
        fb = (bits >> jnp.uint32(9)) | jnp.uint32(0x3F800000)
        u = lax.bitcast_convert_type(fb, jnp.float32) - jnp.float32(1.0)
        u = jnp.maximum(jnp.float32(TINY), u)
        g = -jnp.log(-jnp.log(u))
        scores = g + logits
        m = jnp.max(scores, axis=0, keepdims=True)
        samp = jnp.min(jnp.where(scores == m, pidx, jnp.int32(P)),
                       axis=0, keepdims=True)
        onehot = (pidx == samp).astype(jnp.float32)
        pout_ref[pl.ds(k * B, B), :] = lax.dot_general(
            onehot, table_ref[...], (((0,), (0,)), ((), ())),
            preferred_element_type=jnp.float32)
        pg = jnp.dot(probv_ref[...], onehot,
                     preferred_element_type=jnp.float32)
        e = jnp.exp(pg)
        pn_s[pl.ds(i, 1), :] = jnp.log(e / (ALPHA * e + CMIX))

    @pl.when(c == GRID - 1)
    def _():
        pn = pn_s[...]
        m2 = jnp.max(pn, axis=0, keepdims=True)
        lse = jnp.log(jnp.sum(jnp.exp(pn - m2), axis=0, keepdims=True)) + m2
        pnew_ref[...] = pn - lse


@functools.partial(jax.jit, static_argnames=("interpret",))
def kernel(particles, prob, interpret=False):
    prob_pb = prob.reshape(P, B)
    table = particles[:P]
    probv = prob.reshape(1, -1)[:, :P]
    particles_new, prob_new = pl.pallas_call(
        _resample_kernel,
        grid=(GRID,),
        in_specs=[
            pl.BlockSpec((P, B), lambda c: (0, 0)),
            pl.BlockSpec((P, H), lambda c: (0, 0)),
            pl.BlockSpec((1, P), lambda c: (0, 0)),
        ],
        out_specs=[
            pl.BlockSpec((CHUNK * B, H), lambda c: (c, 0)),
            pl.BlockSpec((P, B), lambda c: (0, 0)),
        ],
        out_shape=[
            jax.ShapeDtypeStruct((P * B, H), jnp.float32),
            jax.ShapeDtypeStruct((P, B), jnp.float32),
        ],
        scratch_shapes=[
            pltpu.VMEM((P, B), jnp.float32),
            pltpu.VMEM((P, B), jnp.float32),
        ],
        interpret=interpret,
    )(prob_pb, table, probv)
    return particles_new, prob_new


# speedup vs baseline: 4.1714x; 1.0456x over previous
import functools

import jax
import jax.numpy as jnp
from jax import lax
from jax.experimental import pallas as pl
from jax.experimental.pallas import tpu as pltpu

P = 128
B = 128
H = 256
ALPHA = 0.5
CMIX = (1.0 - ALPHA) / P
TINY = float(jnp.finfo(jnp.float32).tiny)
CHUNK = 16
GRID = P // CHUNK


_KS1 = 42
_KS2 = 42 ^ 0x1BD11BDA


def _threefry_bits(x1):
    def rounds(x0, x1, rots, first=False):
        for r in rots:
            x0 = x1 if first else x0 + x1
            first = False
            x1 = (x1 << jnp.uint32(r)) | (x1 >> jnp.uint32(32 - r))
            x1 = x1 ^ x0
        return x0, x1

    r0 = (13, 15, 26, 6)
    r1 = (17, 29, 16, 24)
    x0, x1 = rounds(None, x1, r0, first=True)
    x0 = x0 + jnp.uint32(_KS1)
    x1 = x1 + jnp.uint32((_KS2 + 1) & 0xFFFFFFFF)
    x0, x1 = rounds(x0, x1, r1)
    x0 = x0 + jnp.uint32(_KS2)
    x1 = x1 + jnp.uint32(2)
    x0, x1 = rounds(x0, x1, r0)
    x1 = x1 + jnp.uint32(_KS1 + 3)
    x0, x1 = rounds(x0, x1, r1)
    x0 = x0 + jnp.uint32(_KS1)
    x1 = x1 + jnp.uint32((_KS2 + 4) & 0xFFFFFFFF)
    x0, x1 = rounds(x0, x1, r0)
    x0 = x0 + jnp.uint32(_KS2)
    x1 = x1 + jnp.uint32(5)
    return x0 ^ x1


def _resample_kernel(prob_pb_ref, table_ref, probv_ref, pout_ref, pnew_ref,
                     logits_s, pn_s):
    c = pl.program_id(0)

    @pl.when(c == 0)
    def _():
        logits_s[...] = jnp.log(ALPHA * jnp.exp(prob_pb_ref[...]) + CMIX)

    logits = logits_s[...]
    pidx = lax.broadcasted_iota(jnp.int32, (P, B), 0)
    lane_b = lax.broadcasted_iota(jnp.uint32, (P, B), 1)
    subl_p = lax.broadcasted_iota(jnp.uint32, (P, B), 0)
    lbp = lane_b * jnp.uint32(P) + subl_p

    for k in range(CHUNK):
        i = c * CHUNK + k
        x1 = lbp + (jnp.uint32(P * B) * i.astype(jnp.uint32) + jnp.uint32(_KS1))
        bits = _threefry_bits(x1)
        fb = (bits >> jnp.uint32(9)) | jnp.uint32(0x3F800000)
        u = lax.bitcast_convert_type(fb, jnp.float32) - jnp.float32(1.0)
        u = jnp.maximum(jnp.float32(TINY), u)
        g = -jnp.log(-jnp.log(u))
        scores = g + logits
        m = jnp.max(scores, axis=0, keepdims=True)
        samp = jnp.min(jnp.where(scores == m, pidx, jnp.int32(P)),
                       axis=0, keepdims=True)
        onehot = (pidx == samp).astype(jnp.float32)
        pout_ref[pl.ds(k * B, B), :] = lax.dot_general(
            onehot, table_ref[...], (((0,), (0,)), ((), ())),
            preferred_element_type=jnp.float32)
        pg = jnp.dot(probv_ref[...], onehot,
                     preferred_element_type=jnp.float32)
        e = jnp.exp(pg)
        pn_s[pl.ds(i, 1), :] = jnp.log(e / (ALPHA * e + CMIX))

    @pl.when(c == GRID - 1)
    def _():
        pn = pn_s[...]
        m2 = jnp.max(pn, axis=0, keepdims=True)
        lse = jnp.log(jnp.sum(jnp.exp(pn - m2), axis=0, keepdims=True)) + m2
        pnew_ref[...] = pn - lse


@functools.partial(jax.jit, static_argnames=("interpret",))
def kernel(particles, prob, interpret=False):
    prob_pb = prob.reshape(P, B)
    table = particles[:P]
    probv = prob.reshape(1, -1)[:, :P]
    particles_new, prob_new = pl.pallas_call(
        _resample_kernel,
        grid=(GRID,),
        in_specs=[
            pl.BlockSpec((P, B), lambda c: (0, 0)),
            pl.BlockSpec((P, H), lambda c: (0, 0)),
            pl.BlockSpec((1, P), lambda c: (0, 0)),
        ],
        out_specs=[
            pl.BlockSpec((CHUNK * B, H), lambda c: (c, 0)),
            pl.BlockSpec((P, B), lambda c: (0, 0)),
        ],
        out_shape=[
            jax.ShapeDtypeStruct((P * B, H), jnp.float32),
            jax.ShapeDtypeStruct((P, B), jnp.float32),
        ],
        scratch_shapes=[
            pltpu.VMEM((P, B), jnp.float32),
            pltpu.VMEM((P, B), jnp.float32),
        ],
        interpret=interpret,
    )(prob_pb, table, probv)
    return particles_new, prob_new


# speedup vs baseline: 4.2205x; 1.0118x over previous
import functools

import jax
import jax.numpy as jnp
from jax import lax
from jax.experimental import pallas as pl
from jax.experimental.pallas import tpu as pltpu

P = 128
B = 128
H = 256
ALPHA = 0.5
CMIX = (1.0 - ALPHA) / P
TINY = float(jnp.finfo(jnp.float32).tiny)
CHUNK = 32
GRID = P // CHUNK


_KS1 = 42
_KS2 = 42 ^ 0x1BD11BDA


def _threefry_bits(x1):
    def rounds(x0, x1, rots, first=False):
        for r in rots:
            x0 = x1 if first else x0 + x1
            first = False
            x1 = (x1 << jnp.uint32(r)) | (x1 >> jnp.uint32(32 - r))
            x1 = x1 ^ x0
        return x0, x1

    r0 = (13, 15, 26, 6)
    r1 = (17, 29, 16, 24)
    x0, x1 = rounds(None, x1, r0, first=True)
    x0 = x0 + jnp.uint32(_KS1)
    x1 = x1 + jnp.uint32((_KS2 + 1) & 0xFFFFFFFF)
    x0, x1 = rounds(x0, x1, r1)
    x0 = x0 + jnp.uint32(_KS2)
    x1 = x1 + jnp.uint32(2)
    x0, x1 = rounds(x0, x1, r0)
    x1 = x1 + jnp.uint32(_KS1 + 3)
    x0, x1 = rounds(x0, x1, r1)
    x0 = x0 + jnp.uint32(_KS1)
    x1 = x1 + jnp.uint32((_KS2 + 4) & 0xFFFFFFFF)
    x0, x1 = rounds(x0, x1, r0)
    x0 = x0 + jnp.uint32(_KS2)
    x1 = x1 + jnp.uint32(5)
    return x0 ^ x1


def _resample_kernel(prob_pb_ref, table_ref, probv_ref, pout_ref, pnew_ref,
                     logits_s, pn_s):
    c = pl.program_id(0)

    @pl.when(c == 0)
    def _():
        logits_s[...] = jnp.log(ALPHA * jnp.exp(prob_pb_ref[...]) + CMIX)

    logits = logits_s[...]
    pidx = lax.broadcasted_iota(jnp.int32, (P, B), 0)
    lane_b = lax.broadcasted_iota(jnp.uint32, (P, B), 1)
    subl_p = lax.broadcasted_iota(jnp.uint32, (P, B), 0)
    lbp = lane_b * jnp.uint32(P) + subl_p

    for k in range(CHUNK):
        i = c * CHUNK + k
        x1 = lbp + (jnp.uint32(P * B) * i.astype(jnp.uint32) + jnp.uint32(_KS1))
        bits = _threefry_bits(x1)
        fb = (bits >> jnp.uint32(9)) | jnp.uint32(0x3F800000)
        u = lax.bitcast_convert_type(fb, jnp.float32) - jnp.float32(1.0)
        u = jnp.maximum(jnp.float32(TINY), u)
        g = -jnp.log(-jnp.log(u))
        scores = g + logits
        m = jnp.max(scores, axis=0, keepdims=True)
        samp = jnp.min(jnp.where(scores == m, pidx, jnp.int32(P)),
                       axis=0, keepdims=True)
        onehot = (pidx == samp).astype(jnp.float32)
        pout_ref[pl.ds(k * B, B), :] = lax.dot_general(
            onehot, table_ref[...], (((0,), (0,)), ((), ())),
            preferred_element_type=jnp.float32)
        pg = jnp.dot(probv_ref[...], onehot,
                     preferred_element_type=jnp.float32)
        e = jnp.exp(pg)
        pn_s[pl.ds(i, 1), :] = jnp.log(e / (ALPHA * e + CMIX))

    @pl.when(c == GRID - 1)
    def _():
        pn = pn_s[...]
        m2 = jnp.max(pn, axis=0, keepdims=True)
        lse = jnp.log(jnp.sum(jnp.exp(pn - m2), axis=0, keepdims=True)) + m2
        pnew_ref[...] = pn - lse


@functools.partial(jax.jit, static_argnames=("interpret",))
def kernel(particles, prob, interpret=False):
    prob_pb = prob.reshape(P, B)
    table = particles[:P]
    probv = prob.reshape(1, -1)[:, :P]
    particles_new, prob_new = pl.pallas_call(
        _resample_kernel,
        grid=(GRID,),
        in_specs=[
            pl.BlockSpec((P, B), lambda c: (0, 0)),
            pl.BlockSpec((P, H), lambda c: (0, 0)),
            pl.BlockSpec((1, P), lambda c: (0, 0)),
        ],
        out_specs=[
            pl.BlockSpec((CHUNK * B, H), lambda c: (c, 0)),
            pl.BlockSpec((P, B), lambda c: (0, 0)),
        ],
        out_shape=[
            jax.ShapeDtypeStruct((P * B, H), jnp.float32),
            jax.ShapeDtypeStruct((P, B), jnp.float32),
        ],
        scratch_shapes=[
            pltpu.VMEM((P, B), jnp.float32),
            pltpu.VMEM((P, B), jnp.float32),
        ],
        interpret=interpret,
    )(prob_pb, table, probv)
    return particles_new, prob_new
